# baseline (speedup 1.0000x reference)
import functools

import jax
import jax.numpy as jnp
from jax import lax
from jax.experimental import pallas as pl
from jax.experimental.pallas import tpu as pltpu
from jax.experimental.pallas import tpu_sc as plsc

_NUM_CORES = 2
_NUM_SUBCORES = 16
_NUM_WORKERS = _NUM_CORES * _NUM_SUBCORES

_SC_ROWS = 4096
_TC_BLOCK = 512


def _sc_gather(table, idx, batch_out):
    vocab, dim = table.shape
    (s,) = idx.shape
    b_per_w = s // _NUM_WORKERS
    mesh = plsc.VectorSubcoreMesh(core_axis_name="c", subcore_axis_name="s")

    @functools.partial(
        pl.kernel,
        mesh=mesh,
        out_type=jax.ShapeDtypeStruct((s, dim), table.dtype),
        scratch_types=[
            pltpu.VMEM((b_per_w,), jnp.int32),
            pltpu.VMEM((b_per_w, dim), jnp.float32),
            pltpu.SemaphoreType.DMA,
        ],
    )
    def gather_kernel(table_hbm, idx_hbm, out_hbm, idx_v, rows_v, sem):
        wid = lax.axis_index("s") * _NUM_CORES + lax.axis_index("c")
        base = wid * b_per_w
        pltpu.sync_copy(idx_hbm.at[pl.ds(base, b_per_w)], idx_v)
        pltpu.async_copy(table_hbm.at[idx_v], rows_v, sem).wait()
        pltpu.sync_copy(rows_v, out_hbm.at[pl.ds(base, b_per_w)])

    return gather_kernel(table, idx)


def _tc_onehot_matmul_body(idx_ref, table_ref, out_ref):
    vocab = table_ref.shape[0]
    idx = idx_ref[0, 0, :]
    onehot = (idx[:, None] == lax.broadcasted_iota(
        jnp.int32, (_TC_BLOCK, vocab), 1)).astype(jnp.bfloat16)
    out_ref[...] = jnp.dot(onehot, table_ref[...].astype(jnp.bfloat16),
                           preferred_element_type=jnp.float32)


def _tc_lookup_full(t_all, table, sc_rows):
    (batch,) = t_all.shape
    vocab, dim = table.shape
    n_blocks = (batch - sc_rows) // _TC_BLOCK
    first = sc_rows // _TC_BLOCK
    idx3 = t_all.reshape(batch // _TC_BLOCK, 1, _TC_BLOCK)
    return pl.pallas_call(
        _tc_onehot_matmul_body,
        grid=(n_blocks,),
        in_specs=[
            pl.BlockSpec((1, 1, _TC_BLOCK), lambda i: (first + i, 0, 0)),
            pl.BlockSpec((vocab, dim), lambda i: (0, 0)),
        ],
        out_specs=pl.BlockSpec((_TC_BLOCK, dim), lambda i: (first + i, 0)),
        out_shape=jax.ShapeDtypeStruct((batch, dim), jnp.float32),
    )(idx3, table)


def kernel(t, pos_embedding):
    (batch,) = t.shape
    t = t.astype(jnp.int32)
    sc_part = _sc_gather(pos_embedding, t[:_SC_ROWS], batch)
    tc_full = _tc_lookup_full(t, pos_embedding, _SC_ROWS)
    return lax.dynamic_update_slice(tc_full, sc_part, (0, 0))


# speedup vs baseline: 1.6455x; 1.6455x over previous
import functools

import jax
import jax.numpy as jnp
from jax import lax
from jax.experimental import pallas as pl
from jax.experimental.pallas import tpu as pltpu
from jax.experimental.pallas import tpu_sc as plsc

_NUM_CORES = 2
_NUM_SUBCORES = 16
_NUM_WORKERS = _NUM_CORES * _NUM_SUBCORES

_SC_ROWS = 0
_TC_BLOCK = 512


def _sc_gather(table, idx, batch_out):
    vocab, dim = table.shape
    (s,) = idx.shape
    b_per_w = s // _NUM_WORKERS
    mesh = plsc.VectorSubcoreMesh(core_axis_name="c", subcore_axis_name="s")

    @functools.partial(
        pl.kernel,
        mesh=mesh,
        out_type=jax.ShapeDtypeStruct((s, dim), table.dtype),
        scratch_types=[
            pltpu.VMEM((b_per_w,), jnp.int32),
            pltpu.VMEM((b_per_w, dim), jnp.float32),
            pltpu.SemaphoreType.DMA,
        ],
    )
    def gather_kernel(table_hbm, idx_hbm, out_hbm, idx_v, rows_v, sem):
        wid = lax.axis_index("s") * _NUM_CORES + lax.axis_index("c")
        base = wid * b_per_w
        pltpu.sync_copy(idx_hbm.at[pl.ds(base, b_per_w)], idx_v)
        pltpu.async_copy(table_hbm.at[idx_v], rows_v, sem).wait()
        pltpu.sync_copy(rows_v, out_hbm.at[pl.ds(base, b_per_w)])

    return gather_kernel(table, idx)


def _tc_onehot_matmul_body(idx_ref, table_ref, out_ref):
    vocab = table_ref.shape[0]
    idx = idx_ref[0, 0, :]
    onehot = (idx[:, None] == lax.broadcasted_iota(
        jnp.int32, (_TC_BLOCK, vocab), 1)).astype(jnp.bfloat16)
    out_ref[...] = jnp.dot(onehot, table_ref[...].astype(jnp.bfloat16),
                           preferred_element_type=jnp.float32)


def _tc_lookup_full(t_all, table, sc_rows):
    (batch,) = t_all.shape
    vocab, dim = table.shape
    n_blocks = (batch - sc_rows) // _TC_BLOCK
    first = sc_rows // _TC_BLOCK
    idx3 = t_all.reshape(batch // _TC_BLOCK, 1, _TC_BLOCK)
    return pl.pallas_call(
        _tc_onehot_matmul_body,
        grid=(n_blocks,),
        in_specs=[
            pl.BlockSpec((1, 1, _TC_BLOCK), lambda i: (first + i, 0, 0)),
            pl.BlockSpec((vocab, dim), lambda i: (0, 0)),
        ],
        out_specs=pl.BlockSpec((_TC_BLOCK, dim), lambda i: (first + i, 0)),
        out_shape=jax.ShapeDtypeStruct((batch, dim), jnp.float32),
    )(idx3, table)


def kernel(t, pos_embedding):
    (batch,) = t.shape
    t = t.astype(jnp.int32)
    if _SC_ROWS == 0:
        return _tc_lookup_full(t, pos_embedding, 0)
    sc_part = _sc_gather(pos_embedding, t[:_SC_ROWS], batch)
    tc_full = _tc_lookup_full(t, pos_embedding, _SC_ROWS)
    return lax.dynamic_update_slice(tc_full, sc_part, (0, 0))
